# baseline (speedup 1.0000x reference)
import jax
import jax.numpy as jnp
from jax import lax
from jax.experimental import pallas as pl
from jax.experimental.pallas import tpu as pltpu
from jax.experimental.pallas import tpu_sc as plsc

_N_COLS = 26
_VOCAB = 100000
_D = 32
_DP = 128
_B = 16384
_NC, _NS = 2, 16
_NW = _NC * _NS
_BPW = _B // _NW
_NBB = 4
_NSUP = _BPW // (2 * _NBB)
_LANES = 16


def _body(Xp, tp, out, idx1, idxl0, idxl1, rows_v, gsem, isem,
          osem0, osem1):
    wid = lax.axis_index("s") * _NC + lax.axis_index("c")
    lanes = lax.iota(jnp.int32, _LANES)
    off_lo = lanes * _VOCAB
    off_hi = (lanes + _LANES) * _VOCAB
    hi_mask = lanes < (_N_COLS - _LANES)
    idxls = (idxl0, idxl1)
    osems = (osem0, osem1)

    @pl.loop(0, _NSUP)
    def _super(g):
        b0 = wid * _BPW + g * (2 * _NBB)
        for bl in range(2 * _NBB):
            pltpu.async_copy(Xp.at[b0 + bl], idx1.at[bl], isem)
        for bl in range(2 * _NBB):
            pltpu.make_async_copy(Xp.at[0], idx1.at[bl], isem).wait()

        for half in range(2):
            bh = b0 + half * _NBB
            idxl = idxls[half]

            for bl in range(_NBB):
                row = half * _NBB + bl
                rv0 = plsc.bitcast(
                    idx1[row, pl.ds(0, _LANES)], jnp.int32) + off_lo
                rv1 = plsc.bitcast(
                    idx1[row, pl.ds(_LANES, _LANES)], jnp.int32) + off_hi
                idxl[pl.ds(bl * 2 * _LANES, _LANES)] = rv0
                plsc.store_compressed(
                    idxl.at[pl.ds(bl * 2 * _LANES + _LANES, _LANES)],
                    rv1, mask=hi_mask)

            for bl in range(_NBB):
                pltpu.async_copy(
                    tp.at[idxl.at[pl.ds(bl * 2 * _LANES, _N_COLS)]],
                    rows_v.at[half, bl, pl.ds(0, _N_COLS)], gsem)
            for bl in range(_NBB):
                pltpu.make_async_copy(
                    tp.at[idxl.at[pl.ds(bl * 2 * _LANES, _N_COLS)]],
                    rows_v.at[half, bl, pl.ds(0, _N_COLS)], gsem).wait()

            @pl.when(g > 0)
            def _():
                pltpu.make_async_copy(
                    rows_v.at[half, :, pl.ds(0, _N_COLS)],
                    out.at[pl.ds(0, _NBB)], osems[half]).wait()

            pltpu.async_copy(rows_v.at[half, :, pl.ds(0, _N_COLS)],
                             out.at[pl.ds(bh, _NBB)], osems[half])

    for half in range(2):
        pltpu.make_async_copy(
            rows_v.at[half, :, pl.ds(0, _N_COLS)],
            out.at[pl.ds(0, _NBB)], osems[half]).wait()


def kernel(X, table):
    Xp = jax.lax.bitcast_convert_type(
        jnp.pad(X, ((0, 0), (0, 2 * _LANES - _N_COLS))), jnp.float32)
    tp = jnp.pad(table, ((0, 0), (0, _DP - _D)))
    mesh = plsc.VectorSubcoreMesh(
        core_axis_name="c", subcore_axis_name="s",
        num_cores=_NC, num_subcores=_NS)
    scratch = [
        pltpu.VMEM((2 * _NBB, _D), jnp.float32),
        pltpu.VMEM((2 * _NBB * _LANES,), jnp.int32),
        pltpu.VMEM((2 * _NBB * _LANES,), jnp.int32),
        pltpu.VMEM((2, _NBB, 32, _DP), jnp.float32),
        pltpu.SemaphoreType.DMA,
        pltpu.SemaphoreType.DMA,
        pltpu.SemaphoreType.DMA,
        pltpu.SemaphoreType.DMA,
    ]
    out_wide = pl.kernel(
        _body,
        out_type=jax.ShapeDtypeStruct((_B, _N_COLS, _DP), jnp.float32),
        mesh=mesh,
        scratch_types=scratch,
        compiler_params=pltpu.CompilerParams(needs_layout_passes=False),
    )(Xp, tp)
    return out_wide[:, :, :_D]


# speedup vs baseline: 1.3085x; 1.3085x over previous
import jax
import jax.numpy as jnp
from jax import lax
from jax.experimental import pallas as pl
from jax.experimental.pallas import tpu as pltpu
from jax.experimental.pallas import tpu_sc as plsc

_N_COLS = 26
_VOCAB = 100000
_D = 32
_B = 16384
_NC, _NS = 2, 16
_NW = _NC * _NS
_BPW = _B // _NW
_NBB = 4
_NSUP = _BPW // (2 * _NBB)
_LANES = 16


def _body(X, table, out, idx1, rows_v, gsem, isem, osem0, osem1):
    wid = lax.axis_index("s") * _NC + lax.axis_index("c")
    lanes = lax.iota(jnp.int32, _LANES)
    off_lo = lanes * _VOCAB
    off_hi = (lanes + _LANES) * _VOCAB
    osems = (osem0, osem1)

    @pl.loop(0, _NSUP)
    def _super(g):
        b0 = wid * _BPW + g * (2 * _NBB)
        for bl in range(2 * _NBB):
            pltpu.async_copy(X.at[b0 + bl], idx1.at[bl], isem)
        for bl in range(2 * _NBB):
            pltpu.make_async_copy(X.at[0], idx1.at[bl], isem).wait()

        for half in range(2):
            bh = b0 + half * _NBB

            for bl in range(_NBB):
                row = half * _NBB + bl
                rv0 = plsc.bitcast(
                    idx1[row, pl.ds(0, _LANES)], jnp.int32) + off_lo
                rv1 = plsc.bitcast(
                    idx1[row, pl.ds(_LANES, _LANES)], jnp.int32) + off_hi
                for lane in range(_LANES):
                    pltpu.async_copy(
                        table.at[rv0[lane]], rows_v.at[half, bl, lane], gsem)
                for lane in range(_N_COLS - _LANES):
                    pltpu.async_copy(
                        table.at[rv1[lane]], rows_v.at[half, bl, _LANES + lane],
                        gsem)

            for bl in range(_NBB):
                pltpu.make_async_copy(
                    out.at[0], rows_v.at[half, bl], gsem).wait()

            @pl.when(g > 0)
            def _():
                pltpu.make_async_copy(
                    rows_v.at[half], out.at[pl.ds(0, _NBB)], osems[half]).wait()

            pltpu.async_copy(rows_v.at[half], out.at[pl.ds(bh, _NBB)],
                             osems[half])

    for half in range(2):
        pltpu.make_async_copy(
            rows_v.at[half], out.at[pl.ds(0, _NBB)], osems[half]).wait()


def kernel(X, table):
    Xp = jax.lax.bitcast_convert_type(
        jnp.pad(X, ((0, 0), (0, 2 * _LANES - _N_COLS))), jnp.float32)
    mesh = plsc.VectorSubcoreMesh(
        core_axis_name="c", subcore_axis_name="s",
        num_cores=_NC, num_subcores=_NS)
    scratch = [
        pltpu.VMEM((2 * _NBB, _D), jnp.float32),
        pltpu.VMEM((2, _NBB, _N_COLS, _D), jnp.float32),
        pltpu.SemaphoreType.DMA,
        pltpu.SemaphoreType.DMA,
        pltpu.SemaphoreType.DMA,
        pltpu.SemaphoreType.DMA,
    ]
    return pl.kernel(
        _body,
        out_type=jax.ShapeDtypeStruct((_B, _N_COLS, _D), jnp.float32),
        mesh=mesh,
        scratch_types=scratch,
        compiler_params=pltpu.CompilerParams(needs_layout_passes=False),
    )(Xp, table)
